# Initial kernel scaffold; baseline (speedup 1.0000x reference)
import jax
import jax.numpy as jnp
from jax.experimental import pallas as pl


def kernel(x, cb0, cb1, cb2):
    raise NotImplementedError("write your pallas kernel here")


# speedup vs baseline: 1.0751x; 1.0751x over previous
import jax
import jax.numpy as jnp
from jax import lax
from jax.experimental import pallas as pl

_B = 4096
_D = 64
_K = 8192
_COMMITMENT_COST = 0.25

_BB = 256
_KT = 1024


def _argmin_block(r_ref, cb_ref, rn_ref, cn_ref, idx_ref):
    r2 = r_ref[...] * 2.0
    rn = rn_ref[...]
    runmin = jnp.full((_BB,), jnp.inf, dtype=jnp.float32)
    runidx = jnp.zeros((_BB,), dtype=jnp.int32)
    for kt in range(_K // _KT):
        cb_t = cb_ref[kt * _KT:(kt + 1) * _KT, :]
        m2 = lax.dot_general(r2, cb_t, (((1,), (1,)), ((), ())),
                             preferred_element_type=jnp.float32)
        d = (rn - m2) + cn_ref[:, kt * _KT:(kt + 1) * _KT]
        tmin = jnp.min(d, axis=1)
        cols = lax.broadcasted_iota(jnp.int32, (_BB, _KT), 1)
        tidx = jnp.min(jnp.where(d == tmin[:, None], cols, _K), axis=1) + kt * _KT
        upd = tmin < runmin
        runmin = jnp.where(upd, tmin, runmin)
        runidx = jnp.where(upd, tidx, runidx)
    idx_ref[0, 0, :] = runidx


def _argmin_call(residual, cb, rnorm, cnorm):
    idx3 = pl.pallas_call(
        _argmin_block,
        grid=(_B // _BB,),
        in_specs=[
            pl.BlockSpec((_BB, _D), lambda b: (b, 0)),
            pl.BlockSpec((_K, _D), lambda b: (0, 0)),
            pl.BlockSpec((_BB, 1), lambda b: (b, 0)),
            pl.BlockSpec((1, _K), lambda b: (0, 0)),
        ],
        out_specs=pl.BlockSpec((1, 1, _BB), lambda b: (b, 0, 0)),
        out_shape=jax.ShapeDtypeStruct((_B // _BB, 1, _BB), jnp.int32),
    )(residual, cb, rnorm, cnorm)
    return idx3.reshape(_B)


def kernel(x, cb0, cb1, cb2):
    residual = x
    quantized_sum = jnp.zeros_like(x)
    all_indices = []
    total_commitment_loss = jnp.float32(0.0)
    for cb in (cb0, cb1, cb2):
        cnorm = jnp.sum(cb ** 2, axis=1)[None, :]
        rnorm = jnp.sum(residual ** 2, axis=1, keepdims=True)
        idx = _argmin_call(residual, cb, rnorm, cnorm)
        q = jnp.take(cb, idx, axis=0)
        commit = jnp.mean((q - residual) ** 2)
        q_ste = residual + (q - residual)
        all_indices.append(idx)
        quantized_sum = quantized_sum + q_ste
        residual = residual - q_ste
        total_commitment_loss = total_commitment_loss + commit
    reconstruction_loss = jnp.mean((quantized_sum - x) ** 2)
    total_loss = reconstruction_loss + _COMMITMENT_COST * total_commitment_loss
    return (quantized_sum, jnp.stack(all_indices, axis=0),
            reconstruction_loss, total_commitment_loss, total_loss)
